# Initial kernel scaffold; baseline (speedup 1.0000x reference)
import jax
import jax.numpy as jnp
from jax.experimental import pallas as pl


def kernel(boxes, classification):
    raise NotImplementedError("write your pallas kernel here")


# speedup vs baseline: 16.9905x; 16.9905x over previous
import jax
import jax.numpy as jnp
from jax.experimental import pallas as pl
from jax.experimental.pallas import tpu as pltpu

_MAX_DET = 300
_NMS_THR = 0.5
_SCORE_THR = 0.05
_LANES = 128


def _score_kernel(cls_ref, av_ref, lab_ref):
    c = cls_ref[0]
    s = jnp.max(c, axis=-1)
    cio = jax.lax.broadcasted_iota(jnp.int32, c.shape, 1)
    lab = jnp.min(jnp.where(c == s[:, None], cio, jnp.int32(2**30)), axis=-1)
    av_ref[0] = jnp.where(s > _SCORE_THR, s, -jnp.inf)[:, None]
    lab_ref[0] = lab[:, None]


def _nms_kernel(av0, labp, x1p, y1p, x2p, y2p,
                osc, olab, ox1, oy1, ox2, oy2,
                avs, ars, ios):
    B, R, L = av0.shape

    def _r2(op, a, keep=True):
        return op(op(a, axis=2, keepdims=True), axis=1, keepdims=True)

    avs[...] = av0[...]
    x1v = x1p[...]
    y1v = y1p[...]
    x2v = x2p[...]
    y2v = y2p[...]
    labv = labp[...]
    ars[...] = (x2v - x1v) * (y2v - y1v)
    arv = ars[...]
    rio = jax.lax.broadcasted_iota(jnp.int32, (1, R, L), 1)
    cio = jax.lax.broadcasted_iota(jnp.int32, (1, R, L), 2)
    ios[...] = rio * L + cio
    iota = ios[...]

    def body(t, carry):
        av = avs[...]
        m = _r2(jnp.max, av)
        idx = _r2(jnp.min, jnp.where(av == m, iota, jnp.int32(2**30)))
        pick = iota == idx
        bx1 = _r2(jnp.sum, jnp.where(pick, x1v, 0.0))
        by1 = _r2(jnp.sum, jnp.where(pick, y1v, 0.0))
        bx2 = _r2(jnp.sum, jnp.where(pick, x2v, 0.0))
        by2 = _r2(jnp.sum, jnp.where(pick, y2v, 0.0))
        bar = _r2(jnp.sum, jnp.where(pick, arv, 0.0))
        blab = _r2(jnp.sum, jnp.where(pick, labv, 0))
        xx1 = jnp.maximum(bx1, x1v)
        yy1 = jnp.maximum(by1, y1v)
        xx2 = jnp.minimum(bx2, x2v)
        yy2 = jnp.minimum(by2, y2v)
        inter = jnp.maximum(xx2 - xx1, 0.0) * jnp.maximum(yy2 - yy1, 0.0)
        iou = inter / (bar + arv - inter + 1e-8)
        avs[...] = jnp.where((iou > _NMS_THR) | pick, -jnp.inf, av)
        valid = m > -1e30
        cm = jax.lax.broadcasted_iota(jnp.int32, (1, _MAX_DET), 1) == t
        v2 = valid[:, 0, :]
        osc[...] = jnp.where(cm, jnp.where(v2, m[:, 0, :], -1.0), osc[...])
        olab[...] = jnp.where(cm, jnp.where(v2, blab[:, 0, :], -1), olab[...])
        ox1[...] = jnp.where(cm, jnp.where(v2, bx1[:, 0, :], -1.0), ox1[...])
        oy1[...] = jnp.where(cm, jnp.where(v2, by1[:, 0, :], -1.0), oy1[...])
        ox2[...] = jnp.where(cm, jnp.where(v2, bx2[:, 0, :], -1.0), ox2[...])
        oy2[...] = jnp.where(cm, jnp.where(v2, by2[:, 0, :], -1.0), oy2[...])
        return carry

    jax.lax.fori_loop(0, _MAX_DET, body, 0)


def kernel(boxes, classification):
    B, N, C = classification.shape
    R = ((N + _LANES - 1) // _LANES + 7) // 8 * 8
    Np = R * _LANES
    CH = 4000 if N % 4000 == 0 else N

    av, lab = pl.pallas_call(
        _score_kernel,
        grid=(B, N // CH),
        in_specs=[pl.BlockSpec((1, CH, C), lambda b, i: (b, i, 0))],
        out_specs=[pl.BlockSpec((1, CH, 1), lambda b, i: (b, i, 0)),
                   pl.BlockSpec((1, CH, 1), lambda b, i: (b, i, 0))],
        out_shape=[jax.ShapeDtypeStruct((B, N, 1), jnp.float32),
                   jax.ShapeDtypeStruct((B, N, 1), jnp.int32)],
    )(classification)

    pad = ((0, 0), (0, Np - N))
    av = jnp.pad(av[..., 0], pad, constant_values=-jnp.inf).reshape(B, R, _LANES)
    lab = jnp.pad(lab[..., 0], pad).reshape(B, R, _LANES)
    planes = [jnp.pad(boxes[..., i], pad).reshape(B, R, _LANES) for i in range(4)]

    f32 = jnp.float32
    osc, olab, ox1, oy1, ox2, oy2 = pl.pallas_call(
        _nms_kernel,
        out_shape=[jax.ShapeDtypeStruct((B, _MAX_DET), f32),
                   jax.ShapeDtypeStruct((B, _MAX_DET), jnp.int32),
                   jax.ShapeDtypeStruct((B, _MAX_DET), f32),
                   jax.ShapeDtypeStruct((B, _MAX_DET), f32),
                   jax.ShapeDtypeStruct((B, _MAX_DET), f32),
                   jax.ShapeDtypeStruct((B, _MAX_DET), f32)],
        scratch_shapes=[pltpu.VMEM((B, R, _LANES), f32),
                        pltpu.VMEM((B, R, _LANES), f32),
                        pltpu.VMEM((1, R, _LANES), jnp.int32)],
    )(av, lab, *planes)

    out_boxes = jnp.stack([ox1, oy1, ox2, oy2], axis=-1)
    return out_boxes, osc, olab


# speedup vs baseline: 25.1070x; 1.4777x over previous
import jax
import jax.numpy as jnp
from jax.experimental import pallas as pl
from jax.experimental.pallas import tpu as pltpu

_MAX_DET = 300
_NMS_THR = 0.5
_SCORE_THR = 0.05
_LANES = 128


def _score_kernel(cls_ref, av_ref, lab_ref):
    c = cls_ref[0]
    s = jnp.max(c, axis=-1)
    cio = jax.lax.broadcasted_iota(jnp.int32, c.shape, 1)
    lab = jnp.min(jnp.where(c == s[:, None], cio, jnp.int32(2**30)), axis=-1)
    av_ref[0] = jnp.where(s > _SCORE_THR, s, -jnp.inf)[:, None]
    lab_ref[0] = lab[:, None]


def _nms_kernel(av0, labp, x1p, y1p, x2p, y2p,
                osc, olab, ox1, oy1, ox2, oy2,
                avs, ars, ios):
    B, R, L = av0.shape

    def _r2(op, a):
        return op(op(a, axis=1, keepdims=True), axis=2, keepdims=True)

    avs[...] = av0[...]
    x1v = x1p[...]
    y1v = y1p[...]
    x2v = x2p[...]
    y2v = y2p[...]
    labv = labp[...]
    ars[...] = (x2v - x1v) * (y2v - y1v)
    arv = ars[...]
    rio = jax.lax.broadcasted_iota(jnp.int32, (1, R, L), 1)
    cio = jax.lax.broadcasted_iota(jnp.int32, (1, R, L), 2)
    ios[...] = rio * L + cio
    iota = ios[...]

    def body(t, carry):
        av = avs[...]
        m = _r2(jnp.max, av)
        idx = _r2(jnp.min, jnp.where(av == m, iota, jnp.int32(2**30)))
        pick = iota == idx
        bx1 = _r2(jnp.sum, jnp.where(pick, x1v, 0.0))
        by1 = _r2(jnp.sum, jnp.where(pick, y1v, 0.0))
        bx2 = _r2(jnp.sum, jnp.where(pick, x2v, 0.0))
        by2 = _r2(jnp.sum, jnp.where(pick, y2v, 0.0))
        bar = (bx2 - bx1) * (by2 - by1)
        blab = _r2(jnp.sum, jnp.where(pick, labv, 0))
        xx1 = jnp.maximum(bx1, x1v)
        yy1 = jnp.maximum(by1, y1v)
        xx2 = jnp.minimum(bx2, x2v)
        yy2 = jnp.minimum(by2, y2v)
        inter = jnp.maximum(xx2 - xx1, 0.0) * jnp.maximum(yy2 - yy1, 0.0)
        iou = inter / (bar + arv - inter + 1e-8)
        avs[...] = jnp.where((iou > _NMS_THR) | pick, -jnp.inf, av)
        valid = m > -1e30
        cm = jax.lax.broadcasted_iota(jnp.int32, (1, _MAX_DET), 1) == t
        v2 = valid[:, 0, :]
        osc[...] = jnp.where(cm, jnp.where(v2, m[:, 0, :], -1.0), osc[...])
        olab[...] = jnp.where(cm, jnp.where(v2, blab[:, 0, :], -1), olab[...])
        ox1[...] = jnp.where(cm, jnp.where(v2, bx1[:, 0, :], -1.0), ox1[...])
        oy1[...] = jnp.where(cm, jnp.where(v2, by1[:, 0, :], -1.0), oy1[...])
        ox2[...] = jnp.where(cm, jnp.where(v2, bx2[:, 0, :], -1.0), ox2[...])
        oy2[...] = jnp.where(cm, jnp.where(v2, by2[:, 0, :], -1.0), oy2[...])
        return carry

    jax.lax.fori_loop(0, _MAX_DET, body, 0)


def kernel(boxes, classification):
    B, N, C = classification.shape
    R = ((N + _LANES - 1) // _LANES + 7) // 8 * 8
    Np = R * _LANES
    CH = 4000 if N % 4000 == 0 else N

    av, lab = pl.pallas_call(
        _score_kernel,
        grid=(B, N // CH),
        in_specs=[pl.BlockSpec((1, CH, C), lambda b, i: (b, i, 0))],
        out_specs=[pl.BlockSpec((1, CH, 1), lambda b, i: (b, i, 0)),
                   pl.BlockSpec((1, CH, 1), lambda b, i: (b, i, 0))],
        out_shape=[jax.ShapeDtypeStruct((B, N, 1), jnp.float32),
                   jax.ShapeDtypeStruct((B, N, 1), jnp.int32)],
    )(classification)

    pad = ((0, 0), (0, Np - N))
    av = jnp.pad(av[..., 0], pad, constant_values=-jnp.inf).reshape(B, R, _LANES)
    lab = jnp.pad(lab[..., 0], pad).reshape(B, R, _LANES)
    planes = [jnp.pad(boxes[..., i], pad).reshape(B, R, _LANES) for i in range(4)]

    f32 = jnp.float32
    osc, olab, ox1, oy1, ox2, oy2 = pl.pallas_call(
        _nms_kernel,
        out_shape=[jax.ShapeDtypeStruct((B, _MAX_DET), f32),
                   jax.ShapeDtypeStruct((B, _MAX_DET), jnp.int32),
                   jax.ShapeDtypeStruct((B, _MAX_DET), f32),
                   jax.ShapeDtypeStruct((B, _MAX_DET), f32),
                   jax.ShapeDtypeStruct((B, _MAX_DET), f32),
                   jax.ShapeDtypeStruct((B, _MAX_DET), f32)],
        scratch_shapes=[pltpu.VMEM((B, R, _LANES), f32),
                        pltpu.VMEM((B, R, _LANES), f32),
                        pltpu.VMEM((1, R, _LANES), jnp.int32)],
    )(av, lab, *planes)

    out_boxes = jnp.stack([ox1, oy1, ox2, oy2], axis=-1)
    return out_boxes, osc, olab
